# baseline (speedup 1.0000x reference)
import functools

import jax
import jax.numpy as jnp
from jax import lax
from jax.experimental import pallas as pl
from jax.experimental.pallas import tpu as pltpu
from jax.experimental.pallas import tpu_sc as plsc

N = 10000
E = 320000
D = 128
C = 45
H = D // 2

NS = 16
B = 128
K = 2
CPT = 158
E_PAD = NS * CPT * B
ROWS_PER_TILE = 640
N_PAD = NS * ROWS_PER_TILE

_mesh = plsc.VectorSubcoreMesh(core_axis_name="c", subcore_axis_name="s")


def _make_sc_agg(with_deg):
    out_type = [pltpu.MemorySpace.HBM((2, N_PAD, H), jnp.float32)]
    scratch = [
        pltpu.VMEM((CPT + 1, B), jnp.int32),
        pltpu.VMEM((CPT + 1, B), jnp.int32),
        pltpu.VMEM((B, H), jnp.float32),
        pltpu.VMEM((B, H), jnp.float32),
        pltpu.VMEM((B, H), jnp.float32),
        pltpu.VMEM_SHARED((N_PAD, H), jnp.float32),
        pltpu.SemaphoreType.DMA,
        pltpu.SemaphoreType.DMA,
        pltpu.SemaphoreType.DMA,
        pltpu.SemaphoreType.DMA,
    ]
    if with_deg:
        out_type.append(pltpu.MemorySpace.HBM((N_PAD, 16), jnp.float32))
        scratch += [
            pltpu.VMEM((B, 16), jnp.float32),
            pltpu.VMEM((B, 16), jnp.float32),
            pltpu.VMEM_SHARED((N_PAD, 16), jnp.float32),
        ]

    def body(h_hbm, src_hbm, dst_hbm, *rest):
        if with_deg:
            (agg_out, deg_out, src_vm, dst_vm, *mid,
             z16, o16, deg_sh) = rest
        else:
            (agg_out, src_vm, dst_vm, *mid,) = rest
        rows = mid[:2]
        zrow, agg_sh = mid[2], mid[3]
        gsems = mid[4:6]
        ssems = mid[6:8]
        cid = lax.axis_index("c")
        sid = lax.axis_index("s")

        zv = jnp.zeros((16,), jnp.float32)
        ov = jnp.ones((16,), jnp.float32)

        def initb(r, carry):
            for c in range(H // 16):
                zrow[r, pl.ds(c * 16, 16)] = zv
            if with_deg:
                z16[r, pl.ds(0, 16)] = zv
                o16[r, pl.ds(0, 16)] = ov
            return carry

        lax.fori_loop(0, B, initb, 0)

        pltpu.sync_copy(src_hbm.at[sid], src_vm)
        pltpu.sync_copy(dst_hbm.at[sid], dst_vm)

        row0 = sid * ROWS_PER_TILE

        def zb(i, carry):
            b = row0 + i * B
            pltpu.sync_copy(zrow, agg_sh.at[pl.ds(b, B)])
            if with_deg:
                pltpu.sync_copy(z16, deg_sh.at[pl.ds(b, B)])
            return carry

        lax.fori_loop(0, ROWS_PER_TILE // B, zb, 0)
        plsc.subcore_barrier()

        for t in range(2):
            pltpu.async_copy(zrow, agg_sh.at[dst_vm.at[t]], ssems[t],
                             add=True)
            if with_deg:
                pltpu.async_copy(z16, deg_sh.at[dst_vm.at[t]], ssems[t],
                                 add=True)

        def drain(t):
            pltpu.make_async_copy(agg_out.at[cid, pl.ds(0, B)], rows[t],
                                  ssems[t]).wait()
            if with_deg:
                pltpu.make_async_copy(h_hbm.at[cid].at[pl.ds(0, B)],
                                      o16, ssems[t]).wait()

        def gwait(t):
            pltpu.make_async_copy(h_hbm.at[cid].at[pl.ds(0, B)], rows[t],
                                  gsems[t]).wait()

        def scat(t, i):
            pltpu.async_copy(rows[t], agg_sh.at[dst_vm.at[i]],
                             ssems[t], add=True)
            if with_deg:
                pltpu.async_copy(o16, deg_sh.at[dst_vm.at[i]],
                                 ssems[t], add=True)

        pltpu.async_copy(h_hbm.at[cid].at[src_vm.at[0]], rows[0], gsems[0])

        def group(j, carry):
            i0 = j * 2
            drain(1)
            pltpu.async_copy(h_hbm.at[cid].at[src_vm.at[i0 + 1]],
                             rows[1], gsems[1])
            gwait(0)
            scat(0, i0)
            gwait(1)
            scat(1, i0 + 1)
            drain(0)
            pltpu.async_copy(h_hbm.at[cid].at[src_vm.at[i0 + 2]],
                             rows[0], gsems[0])
            return carry

        lax.fori_loop(0, CPT // 2, group, 0)
        gwait(0)
        drain(0)
        drain(1)
        plsc.subcore_barrier()

        def ob(i, carry):
            b = row0 + i * B
            pltpu.sync_copy(agg_sh.at[pl.ds(b, B)], agg_out.at[cid, pl.ds(b, B)])
            return carry

        lax.fori_loop(0, ROWS_PER_TILE // B, ob, 0)
        if with_deg:
            @pl.when(cid == 0)
            def _():
                pltpu.sync_copy(deg_sh.at[pl.ds(row0, ROWS_PER_TILE)],
                                deg_out.at[pl.ds(row0, ROWS_PER_TILE)])

    return pl.kernel(
        body, mesh=_mesh, out_type=out_type, scratch_types=scratch,
        compiler_params=pltpu.CompilerParams(use_tc_tiling_on_sc=False))


_sc_agg_deg = _make_sc_agg(True)
_sc_agg = _make_sc_agg(False)


BN = 1000


def _elu(z):
    return jnp.where(z > 0, z, jnp.exp(jnp.minimum(z, 0.0)) - 1.0)


def _layer_body(h_ref, agg_ref, deg_ref, ws_ref, wn_ref, out_ref):
    h = jnp.concatenate([h_ref[0], h_ref[1]], axis=1)
    agg = jnp.concatenate([agg_ref[0], agg_ref[1]], axis=1)
    deg = deg_ref[...]
    mean = agg / jnp.maximum(deg, 1.0)
    z = (jnp.dot(h, ws_ref[...], preferred_element_type=jnp.float32)
         + jnp.dot(mean, wn_ref[...], preferred_element_type=jnp.float32))
    z = _elu(z)
    out_ref[0] = z[:, :H]
    out_ref[1] = z[:, H:]


def _layer(h_split, aggp, degp, Ws, Wn):
    return pl.pallas_call(
        _layer_body,
        grid=(N // BN,),
        in_specs=[
            pl.BlockSpec((2, BN, H), lambda i: (0, i, 0)),
            pl.BlockSpec((2, BN, H), lambda i: (0, i, 0)),
            pl.BlockSpec((BN, 1), lambda i: (i, 0)),
            pl.BlockSpec((D, D), lambda i: (0, 0)),
            pl.BlockSpec((D, D), lambda i: (0, 0)),
        ],
        out_specs=pl.BlockSpec((2, BN, H), lambda i: (0, i, 0)),
        out_shape=jax.ShapeDtypeStruct((2, N, H), jnp.float32),
    )(h_split, aggp, degp, Ws, Wn)


def _head_body(h_ref, wp_ref, bp_ref, w1_ref, b1_ref, w2_ref, b2_ref, tgt_ref,
               loss_ref, pred_ref):
    h = jnp.concatenate([h_ref[0], h_ref[1]], axis=1)
    emb = jnp.dot(h, wp_ref[...], preferred_element_type=jnp.float32) + bp_ref[...]
    t = _elu(jnp.dot(emb, w1_ref[...], preferred_element_type=jnp.float32) + b1_ref[...])
    logits = jnp.dot(t, w2_ref[...], preferred_element_type=jnp.float32) + b2_ref[...]
    col = lax.broadcasted_iota(jnp.int32, (BN, 128), 1)
    lm = jnp.where(col < C, logits, jnp.float32(-1e30))
    m = jnp.max(lm, axis=1, keepdims=True)
    lse = jnp.log(jnp.sum(jnp.exp(lm - m), axis=1, keepdims=True))
    tgt = tgt_ref[...]
    val = jnp.sum(jnp.where(col == tgt, lm, 0.0), axis=1, keepdims=True)
    part = -jnp.sum(val - m - lse, keepdims=True) * (1.0 / N)

    @pl.when(pl.program_id(0) == 0)
    def _():
        loss_ref[...] = jnp.zeros((1, 1), jnp.float32)

    loss_ref[...] += part
    pred_ref[...] = jnp.min(jnp.where(lm == m, col, 128), axis=1, keepdims=True)


def _head(h2_split, Wp, bp, W1, b1, W2p, b2p, tgt):
    return pl.pallas_call(
        _head_body,
        grid=(N // BN,),
        in_specs=[
            pl.BlockSpec((2, BN, H), lambda i: (0, i, 0)),
            pl.BlockSpec((D, D), lambda i: (0, 0)),
            pl.BlockSpec((1, D), lambda i: (0, 0)),
            pl.BlockSpec((D, D), lambda i: (0, 0)),
            pl.BlockSpec((1, D), lambda i: (0, 0)),
            pl.BlockSpec((D, 128), lambda i: (0, 0)),
            pl.BlockSpec((1, 128), lambda i: (0, 0)),
            pl.BlockSpec((BN, 1), lambda i: (i, 0)),
        ],
        out_specs=[
            pl.BlockSpec((1, 1), lambda i: (0, 0)),
            pl.BlockSpec((BN, 1), lambda i: (i, 0)),
        ],
        out_shape=[
            jax.ShapeDtypeStruct((1, 1), jnp.float32),
            jax.ShapeDtypeStruct((N, 1), jnp.int32),
        ],
    )(h2_split, Wp, bp, W1, b1, W2p, b2p, tgt)


def kernel(x, edge_index, tgt_tags, Ws1, Wn1, Ws2, Wn2, Wp, bp, W1, b1, W2, b2):
    src = edge_index[0]
    dst = edge_index[1]
    pad = E_PAD - E
    src_p = jnp.concatenate([src, jnp.zeros((pad,), jnp.int32)]).reshape(NS, CPT, B)
    pad_dst = N + (jnp.arange(pad, dtype=jnp.int32) % (N_PAD - N - 8))
    dst_p = jnp.concatenate([dst, pad_dst]).reshape(NS, CPT, B)
    src_p = jnp.concatenate([src_p, jnp.zeros((NS, 1, B), jnp.int32)], axis=1)
    dst_p = jnp.concatenate([dst_p, jnp.full((NS, 1, B), N, jnp.int32)], axis=1)
    x_split = x.reshape(N, 2, H).transpose(1, 0, 2)

    aggp1, degp = _sc_agg_deg(x_split, src_p, dst_p)
    deg2 = degp[:N, 0:1]
    h1 = _layer(x_split, aggp1[:, :N], deg2, Ws1, Wn1)
    aggp2 = _sc_agg(h1, src_p, dst_p)
    if isinstance(aggp2, (list, tuple)):
        aggp2 = aggp2[0]
    h2 = _layer(h1, aggp2[:, :N], deg2, Ws2, Wn2)

    W2p = jnp.pad(W2, ((0, 0), (0, 128 - C)))
    b2p = jnp.pad(b2, (0, 128 - C))
    loss_m, pred_m = _head(h2, Wp, bp.reshape(1, D), W1, b1.reshape(1, D),
                           W2p, b2p.reshape(1, 128), tgt_tags.reshape(N, 1))
    return loss_m[0, 0], pred_m.reshape(N)


# speedup vs baseline: 1.0941x; 1.0941x over previous
import functools

import jax
import jax.numpy as jnp
from jax import lax
from jax.experimental import pallas as pl
from jax.experimental.pallas import tpu as pltpu
from jax.experimental.pallas import tpu_sc as plsc

N = 10000
E = 320000
D = 128
C = 45
H = D // 2

NS = 16
B = 128
K = 2
CPT = 158
E_PAD = NS * CPT * B
ROWS_PER_TILE = 640
N_PAD = NS * ROWS_PER_TILE

_mesh = plsc.VectorSubcoreMesh(core_axis_name="c", subcore_axis_name="s")


def _make_sc_agg(with_deg):
    out_type = [pltpu.MemorySpace.HBM((2, N_PAD, H), jnp.float32)]
    scratch = [
        pltpu.VMEM((CPT + 1, B), jnp.int32),
        pltpu.VMEM((CPT + 1, B), jnp.int32),
        pltpu.VMEM((B, H), jnp.float32),
        pltpu.VMEM((B, H), jnp.float32),
        pltpu.VMEM((B, H), jnp.float32),
        pltpu.VMEM_SHARED((N_PAD, H), jnp.float32),
        pltpu.SemaphoreType.DMA,
        pltpu.SemaphoreType.DMA,
        pltpu.SemaphoreType.DMA,
        pltpu.SemaphoreType.DMA,
    ]
    if with_deg:
        out_type.append(pltpu.MemorySpace.HBM((N_PAD, 16), jnp.float32))
        scratch += [
            pltpu.VMEM((B, 16), jnp.float32),
            pltpu.VMEM((B, 16), jnp.float32),
            pltpu.VMEM_SHARED((N_PAD, 16), jnp.float32),
        ]

    def body(h_hbm, src_hbm, dst_hbm, *rest):
        if with_deg:
            (agg_out, deg_out, src_vm, dst_vm, *mid,
             z16, o16, deg_sh) = rest
        else:
            (agg_out, src_vm, dst_vm, *mid,) = rest
        rows = mid[:2]
        zrow, agg_sh = mid[2], mid[3]
        gsems = mid[4:6]
        ssems = mid[6:8]
        cid = lax.axis_index("c")
        sid = lax.axis_index("s")

        zv = jnp.zeros((16,), jnp.float32)
        ov = jnp.ones((16,), jnp.float32)

        def initb(r, carry):
            for c in range(H // 16):
                zrow[r, pl.ds(c * 16, 16)] = zv
            if with_deg:
                z16[r, pl.ds(0, 16)] = zv
                o16[r, pl.ds(0, 16)] = ov
            return carry

        lax.fori_loop(0, B, initb, 0)

        pltpu.sync_copy(src_hbm.at[sid], src_vm)
        pltpu.sync_copy(dst_hbm.at[sid], dst_vm)

        row0 = sid * ROWS_PER_TILE

        def zb(i, carry):
            b = row0 + i * B
            pltpu.sync_copy(zrow, agg_sh.at[pl.ds(b, B)])
            if with_deg:
                pltpu.sync_copy(z16, deg_sh.at[pl.ds(b, B)])
            return carry

        lax.fori_loop(0, ROWS_PER_TILE // B, zb, 0)
        plsc.subcore_barrier()

        for t in range(2):
            pltpu.async_copy(zrow, agg_sh.at[dst_vm.at[t]], ssems[t],
                             add=True)
            if with_deg:
                pltpu.async_copy(z16, deg_sh.at[dst_vm.at[t]], ssems[t],
                                 add=True)

        def drain(t):
            pltpu.make_async_copy(agg_out.at[cid, pl.ds(0, B)], rows[t],
                                  ssems[t]).wait()
            if with_deg:
                pltpu.make_async_copy(h_hbm.at[cid].at[pl.ds(0, B)],
                                      o16, ssems[t]).wait()

        def group(j, carry):
            i0 = j * 2
            for t in range(2):
                drain(t)
                pltpu.async_copy(h_hbm.at[cid].at[src_vm.at[i0 + t]],
                                 rows[t], gsems[0]).wait()
                pltpu.async_copy(rows[t], agg_sh.at[dst_vm.at[i0 + t]],
                                 ssems[t], add=True)
                if with_deg:
                    pltpu.async_copy(o16, deg_sh.at[dst_vm.at[i0 + t]],
                                     ssems[t], add=True)
            return carry

        lax.fori_loop(0, CPT // 2, group, 0)
        drain(0)
        drain(1)
        plsc.subcore_barrier()

        def ob(i, carry):
            b = row0 + i * B
            pltpu.sync_copy(agg_sh.at[pl.ds(b, B)], agg_out.at[cid, pl.ds(b, B)])
            return carry

        lax.fori_loop(0, ROWS_PER_TILE // B, ob, 0)
        if with_deg:
            @pl.when(cid == 0)
            def _():
                pltpu.sync_copy(deg_sh.at[pl.ds(row0, ROWS_PER_TILE)],
                                deg_out.at[pl.ds(row0, ROWS_PER_TILE)])

    return pl.kernel(
        body, mesh=_mesh, out_type=out_type, scratch_types=scratch,
        compiler_params=pltpu.CompilerParams(use_tc_tiling_on_sc=False))


_sc_agg_deg = _make_sc_agg(True)
_sc_agg = _make_sc_agg(False)


BN = 1000


def _elu(z):
    return jnp.where(z > 0, z, jnp.exp(jnp.minimum(z, 0.0)) - 1.0)


def _layer_body(h_ref, agg_ref, deg_ref, ws_ref, wn_ref, out_ref):
    h = jnp.concatenate([h_ref[0], h_ref[1]], axis=1)
    agg = jnp.concatenate([agg_ref[0], agg_ref[1]], axis=1)
    deg = deg_ref[:, 0:1]
    mean = agg / jnp.maximum(deg, 1.0)
    z = (jnp.dot(h, ws_ref[...], preferred_element_type=jnp.float32)
         + jnp.dot(mean, wn_ref[...], preferred_element_type=jnp.float32))
    z = _elu(z)
    out_ref[0] = z[:, :H]
    out_ref[1] = z[:, H:]


def _layer(h_split, aggp, degp, Ws, Wn):
    return pl.pallas_call(
        _layer_body,
        grid=(N // BN,),
        in_specs=[
            pl.BlockSpec((2, BN, H), lambda i: (0, i, 0)),
            pl.BlockSpec((2, BN, H), lambda i: (0, i, 0)),
            pl.BlockSpec((BN, 16), lambda i: (i, 0)),
            pl.BlockSpec((D, D), lambda i: (0, 0)),
            pl.BlockSpec((D, D), lambda i: (0, 0)),
        ],
        out_specs=pl.BlockSpec((2, BN, H), lambda i: (0, i, 0)),
        out_shape=jax.ShapeDtypeStruct((2, N, H), jnp.float32),
    )(h_split, aggp, degp, Ws, Wn)


def _head_body(h_ref, wp_ref, bp_ref, w1_ref, b1_ref, w2_ref, b2_ref, tgt_ref,
               loss_ref, pred_ref):
    h = jnp.concatenate([h_ref[0], h_ref[1]], axis=1)
    emb = jnp.dot(h, wp_ref[...], preferred_element_type=jnp.float32) + bp_ref[...]
    t = _elu(jnp.dot(emb, w1_ref[...], preferred_element_type=jnp.float32) + b1_ref[...])
    logits = jnp.dot(t, w2_ref[...], preferred_element_type=jnp.float32) + b2_ref[...]
    col = lax.broadcasted_iota(jnp.int32, (BN, 128), 1)
    lm = jnp.where(col < C, logits, jnp.float32(-1e30))
    m = jnp.max(lm, axis=1, keepdims=True)
    lse = jnp.log(jnp.sum(jnp.exp(lm - m), axis=1, keepdims=True))
    tgt = tgt_ref[...]
    val = jnp.sum(jnp.where(col == tgt, lm, 0.0), axis=1, keepdims=True)
    part = -jnp.sum(val - m - lse, keepdims=True) * (1.0 / N)

    @pl.when(pl.program_id(0) == 0)
    def _():
        loss_ref[...] = jnp.zeros((1, 1), jnp.float32)

    loss_ref[...] += part
    pred_ref[...] = jnp.min(jnp.where(lm == m, col, 128), axis=1, keepdims=True)


def _head(h2_split, Wp, bp, W1, b1, W2p, b2p, tgt):
    return pl.pallas_call(
        _head_body,
        grid=(N // BN,),
        in_specs=[
            pl.BlockSpec((2, BN, H), lambda i: (0, i, 0)),
            pl.BlockSpec((D, D), lambda i: (0, 0)),
            pl.BlockSpec((1, D), lambda i: (0, 0)),
            pl.BlockSpec((D, D), lambda i: (0, 0)),
            pl.BlockSpec((1, D), lambda i: (0, 0)),
            pl.BlockSpec((D, 128), lambda i: (0, 0)),
            pl.BlockSpec((1, 128), lambda i: (0, 0)),
            pl.BlockSpec((BN, 1), lambda i: (i, 0)),
        ],
        out_specs=[
            pl.BlockSpec((1, 1), lambda i: (0, 0)),
            pl.BlockSpec((BN, 1), lambda i: (i, 0)),
        ],
        out_shape=[
            jax.ShapeDtypeStruct((1, 1), jnp.float32),
            jax.ShapeDtypeStruct((N, 1), jnp.int32),
        ],
    )(h2_split, Wp, bp, W1, b1, W2p, b2p, tgt)


def kernel(x, edge_index, tgt_tags, Ws1, Wn1, Ws2, Wn2, Wp, bp, W1, b1, W2, b2):
    src = edge_index[0]
    dst = edge_index[1]
    pad = E_PAD - E
    src_p = jnp.concatenate([src, jnp.zeros((pad,), jnp.int32)]).reshape(NS, CPT, B)
    pad_dst = N + (jnp.arange(pad, dtype=jnp.int32) % (N_PAD - N - 8))
    dst_p = jnp.concatenate([dst, pad_dst]).reshape(NS, CPT, B)
    src_p = jnp.concatenate([src_p, jnp.zeros((NS, 1, B), jnp.int32)], axis=1)
    dst_p = jnp.concatenate([dst_p, jnp.full((NS, 1, B), N, jnp.int32)], axis=1)
    x_split = x.reshape(N, 2, H).transpose(1, 0, 2)

    aggp1, degp = _sc_agg_deg(x_split, src_p, dst_p)
    h1 = _layer(x_split, aggp1, degp, Ws1, Wn1)
    aggp2 = _sc_agg(h1, src_p, dst_p)
    if isinstance(aggp2, (list, tuple)):
        aggp2 = aggp2[0]
    h2 = _layer(h1, aggp2, degp, Ws2, Wn2)

    W2p = jnp.pad(W2, ((0, 0), (0, 128 - C)))
    b2p = jnp.pad(b2, (0, 128 - C))
    loss_m, pred_m = _head(h2, Wp, bp.reshape(1, D), W1, b1.reshape(1, D),
                           W2p, b2p.reshape(1, 128), tgt_tags.reshape(N, 1))
    return loss_m[0, 0], pred_m.reshape(N)
